# baseline (speedup 1.0000x reference)
import jax
import jax.numpy as jnp
from jax import lax
from jax.experimental import pallas as pl
from jax.experimental.pallas import tpu as pltpu
from jax.experimental.pallas import tpu_sc as plsc

N_NODES = 10000
N_EDGES = 320000
D_IN = 128
D_HID = 128
N_CLS = 16

NC = 2
NS = 16
L = 16
NW = NC * NS
EPW = N_EDGES // NW
NPAD = 10240
RPT = NPAD // NS


def _make_edge_kernel(D, CH, UNROLL):
    W = D + L
    NCHD = D // L
    NCHUNK = EPW // CH
    mesh = plsc.VectorSubcoreMesh(
        core_axis_name="c", subcore_axis_name="s",
        num_cores=NC, num_subcores=NS)

    def body(xl_hbm, xr_hbm, src_hbm, dst_hbm, att_hbm, acc_hbm,
             idxs0, idxd0, xls0, stage0,
             idxs1, idxd1, xls1, stage1,
             attv, accsh, si0, sl0, sr0, si1, sl1, sr1):
        cid = lax.axis_index("c")
        sid = lax.axis_index("s")
        wid = sid * NC + cid

        bufs = [(idxs0, idxd0, xls0, stage0, si0, sl0, sr0),
                (idxs1, idxd1, xls1, stage1, si1, sl1, sr1)]

        zero = jnp.zeros((L,), jnp.float32)

        def zrow(r, carry):
            for c in range(W // L):
                stage0[r, pl.ds(c * L, L)] = zero
            return carry
        lax.fori_loop(0, CH, zrow, 0)

        for b in range(RPT // CH):
            pltpu.sync_copy(stage0, accsh.at[pl.ds(sid * RPT + b * CH, CH)])
        pltpu.sync_copy(att_hbm, attv)
        plsc.subcore_barrier()

        att_c = [attv[pl.ds(c * L, L)] for c in range(NCHD)]

        def sd_start(i, b):
            idxs, idxd, _, _, si, _, _ = bufs[b]
            row = wid * NCHUNK + i
            pltpu.make_async_copy(src_hbm.at[row], idxs, si).start()
            pltpu.make_async_copy(dst_hbm.at[row], idxd, si).start()

        def sd_wait(b):
            idxs, idxd, _, _, si, _, _ = bufs[b]
            pltpu.make_async_copy(src_hbm.at[0], idxs, si).wait()
            pltpu.make_async_copy(dst_hbm.at[0], idxd, si).wait()

        def g1_start(b):
            idxs, _, xls, _, _, sl, _ = bufs[b]
            pltpu.async_copy(xl_hbm.at[idxs], xls, sl)

        def g1_wait(b):
            _, _, xls, _, _, sl, _ = bufs[b]
            pltpu.make_async_copy(xl_hbm.at[bufs[b][0]], xls, sl).wait()

        def g2_start(b):
            _, idxd, xls, _, _, _, sr = bufs[b]
            pltpu.async_copy(xr_hbm.at[idxd], xls, sr, add=True)

        def g2_wait(b):
            _, idxd, xls, _, _, _, sr = bufs[b]
            pltpu.make_async_copy(xr_hbm.at[idxd], xls, sr).wait()

        def process(b):
            _, idxd, xls, stage, _, _, _ = bufs[b]

            @plsc.parallel_loop(0, CH, unroll=UNROLL)
            def edge(e):
                acc = jnp.zeros((L,), jnp.float32)
                s_ch = []
                for c in range(NCHD):
                    sv = xls[e, pl.ds(c * L, L)]
                    lr = jnp.maximum(sv, 0.2 * sv)
                    acc = acc + lr * att_c[c]
                    s_ch.append(sv)
                tot = plsc.cumsum(acc)[L - 1]
                wv = jnp.exp(jnp.broadcast_to(tot, (L,)))
                for c in range(NCHD):
                    stage[e, pl.ds(c * L, L)] = s_ch[c] * wv
                stage[e, pl.ds(D, L)] = wv

            pltpu.sync_copy(stage, accsh.at[idxd], add=True)

        sd_start(0, 0)
        sd_wait(0)
        g1_start(0)
        g1_wait(0)
        g2_start(0)
        sd_start(1, 1)

        def chunk2(i2, carry):
            for b in range(2):
                i = i2 * 2 + b
                g2_wait(b)
                sd_wait(1 - b)
                g1_start(1 - b)
                process(b)
                sd_start(i + 2, b)
                g1_wait(1 - b)
                g2_start(1 - b)
            return carry
        if NCHUNK % 2 == 0:
            lax.fori_loop(0, (NCHUNK - 2) // 2, chunk2, 0)
            tail = [(0, None), (1, None)]
        else:
            lax.fori_loop(0, (NCHUNK - 3) // 2, chunk2, 0)
            tail = [(0, NCHUNK - 1), (1, None), (0, None)]
        for k, (b, nxt) in enumerate(tail):
            g2_wait(b)
            if k + 1 < len(tail):
                sd_wait(1 - b)
                g1_start(1 - b)
            process(b)
            if nxt is not None:
                sd_start(nxt, b)
            if k + 1 < len(tail):
                g1_wait(1 - b)
                g2_start(1 - b)

        plsc.subcore_barrier()
        r0 = sid * RPT
        pltpu.sync_copy(accsh.at[pl.ds(r0, RPT)],
                        acc_hbm.at[cid, pl.ds(r0, RPT)])

    return pl.kernel(
        body,
        out_type=jax.ShapeDtypeStruct((NC, NPAD, W), jnp.float32),
        mesh=mesh,
        scratch_types=[
            pltpu.VMEM((CH,), jnp.int32),
            pltpu.VMEM((CH,), jnp.int32),
            pltpu.VMEM((CH, D), jnp.float32),
            pltpu.VMEM((CH, W), jnp.float32),
            pltpu.VMEM((CH,), jnp.int32),
            pltpu.VMEM((CH,), jnp.int32),
            pltpu.VMEM((CH, D), jnp.float32),
            pltpu.VMEM((CH, W), jnp.float32),
            pltpu.VMEM((D,), jnp.float32),
            pltpu.VMEM_SHARED((NPAD, W), jnp.float32),
            pltpu.SemaphoreType.DMA,
            pltpu.SemaphoreType.DMA,
            pltpu.SemaphoreType.DMA,
            pltpu.SemaphoreType.DMA,
            pltpu.SemaphoreType.DMA,
            pltpu.SemaphoreType.DMA,
        ],
        compiler_params=pltpu.CompilerParams(
            needs_layout_passes=False, use_tc_tiling_on_sc=False),
    )


_edge_conv1 = _make_edge_kernel(D_HID, 40, 4)
_edge_conv2 = _make_edge_kernel(N_CLS, 80, 8)

BR = 1000
_GRID = N_NODES // BR
_HI = lax.Precision.HIGHEST


def _mmT(a, w):
    return lax.dot_general(a, w, (((1,), (1,)), ((), ())), precision=_HI)


def _tc_pre_body(x_ref, wl_ref, bl_ref, wr_ref, br_ref, wlin_ref, blin_ref,
                 xl_ref, xr_ref, lin_ref):
    xb = x_ref[...]
    xl_ref[...] = _mmT(xb, wl_ref[...]) + bl_ref[...]
    xr_ref[...] = _mmT(xb, wr_ref[...]) + br_ref[...]
    lin_ref[...] = _mmT(xb, wlin_ref[...]) + blin_ref[...]


def _tc_mid_body(acc_ref, xl_ref, xr_ref, lin_ref, att_ref, bias_ref,
                 blin_ref, wl2_ref, bl2_ref, wr2_ref, br2_ref,
                 h_ref, xl2_ref, xr2_ref):
    xl = xl_ref[...]
    xr = xr_ref[...]
    s = xl + xr
    lr = jnp.maximum(s, 0.2 * s)
    es = jnp.sum(lr * att_ref[...], axis=1, keepdims=True)
    es = jnp.exp(es)
    acc = acc_ref[...]
    dene = acc[0, :, D_HID:D_HID + 1] + acc[1, :, D_HID:D_HID + 1]
    num = acc[0, :, :D_HID] + acc[1, :, :D_HID] - xr * dene + es * xl
    den = dene + es + 1e-16
    conv = num / den + bias_ref[...]
    z = conv + lin_ref[...] + blin_ref[...]
    h = jnp.where(z > 0, z, jnp.exp(z) - 1.0)
    h_ref[...] = h
    xl2_ref[...] = _mmT(h, wl2_ref[...]) + bl2_ref[...]
    xr2_ref[...] = _mmT(h, wr2_ref[...]) + br2_ref[...]


def _tc_post_body(acc_ref, xl2_ref, xr2_ref, h_ref, att_ref, bias_ref,
                  wlin_ref, blin_ref, out_ref):
    xl = xl2_ref[...]
    xr = xr2_ref[...]
    s = xl + xr
    lr = jnp.maximum(s, 0.2 * s)
    es = jnp.exp(jnp.sum(lr * att_ref[...], axis=1, keepdims=True))
    acc = acc_ref[...]
    dene = acc[0, :, N_CLS:N_CLS + 1] + acc[1, :, N_CLS:N_CLS + 1]
    num = acc[0, :, :N_CLS] + acc[1, :, :N_CLS] - xr * dene + es * xl
    den = dene + es + 1e-16
    o = num / den + bias_ref[...] + _mmT(h_ref[...], wlin_ref[...]) + blin_ref[...]
    m = jnp.max(o, axis=1, keepdims=True)
    o = o - m
    out_ref[...] = o - jnp.log(jnp.sum(jnp.exp(o), axis=1, keepdims=True))


def _full(shape):
    return pl.BlockSpec(shape, lambda i: tuple(0 for _ in shape))


def _rows(d):
    return pl.BlockSpec((BR, d), lambda i: (i, 0))


def _acc_spec(w):
    return pl.BlockSpec((NC, BR, w), lambda i: (0, i, 0))


def kernel(x, edge_index, Wl1, bl1, Wr1, br1, att1, bias1, W_lin1, b_lin1,
           Wl2, bl2, Wr2, br2, att2, bias2, W_lin2, b_lin2):
    src = edge_index[0].astype(jnp.int32)
    dst = edge_index[1].astype(jnp.int32)
    src40 = src.reshape(-1, 40)
    dst40 = dst.reshape(-1, 40)
    src80 = src.reshape(-1, 80)
    dst80 = dst.reshape(-1, 80)


    xl1, xr1, lin1 = pl.pallas_call(
        _tc_pre_body,
        grid=(_GRID,),
        in_specs=[_rows(D_IN), _full((D_HID, D_IN)), _full((1, D_HID)),
                  _full((D_HID, D_IN)), _full((1, D_HID)),
                  _full((D_HID, D_IN)), _full((1, D_HID))],
        out_specs=[_rows(D_HID), _rows(D_HID), _rows(D_HID)],
        out_shape=[jax.ShapeDtypeStruct((N_NODES, D_HID), jnp.float32)] * 3,
    )(x, Wl1, bl1.reshape(1, -1), Wr1, br1.reshape(1, -1),
      W_lin1, b_lin1.reshape(1, -1))

    acc1 = _edge_conv1(xl1, xr1, src40, dst40, att1)

    h, xl2, xr2 = pl.pallas_call(
        _tc_mid_body,
        grid=(_GRID,),
        in_specs=[_acc_spec(D_HID + L), _rows(D_HID), _rows(D_HID),
                  _rows(D_HID), _full((1, D_HID)), _full((1, D_HID)),
                  _full((1, D_HID)), _full((N_CLS, D_HID)), _full((1, N_CLS)),
                  _full((N_CLS, D_HID)), _full((1, N_CLS))],
        out_specs=[_rows(D_HID), _rows(N_CLS), _rows(N_CLS)],
        out_shape=[jax.ShapeDtypeStruct((N_NODES, D_HID), jnp.float32),
                   jax.ShapeDtypeStruct((N_NODES, N_CLS), jnp.float32),
                   jax.ShapeDtypeStruct((N_NODES, N_CLS), jnp.float32)],
    )(acc1, xl1, xr1, lin1, att1.reshape(1, -1), bias1.reshape(1, -1),
      b_lin1.reshape(1, -1), Wl2, bl2.reshape(1, -1), Wr2, br2.reshape(1, -1))

    acc2 = _edge_conv2(xl2, xr2, src80, dst80, att2)

    out = pl.pallas_call(
        _tc_post_body,
        grid=(_GRID,),
        in_specs=[_acc_spec(N_CLS + L), _rows(N_CLS), _rows(N_CLS),
                  _rows(D_HID), _full((1, N_CLS)), _full((1, N_CLS)),
                  _full((N_CLS, D_HID)), _full((1, N_CLS))],
        out_specs=[_rows(N_CLS)],
        out_shape=[jax.ShapeDtypeStruct((N_NODES, N_CLS), jnp.float32)],
    )(acc2, xl2, xr2, h, att2.reshape(1, -1), bias2.reshape(1, -1),
      W_lin2, b_lin2.reshape(1, -1))[0]

    return (out, edge_index)


# speedup vs baseline: 1.3193x; 1.3193x over previous
import jax
import jax.numpy as jnp
from jax import lax
from jax.experimental import pallas as pl
from jax.experimental.pallas import tpu as pltpu
from jax.experimental.pallas import tpu_sc as plsc

N_NODES = 10000
N_EDGES = 320000
D_IN = 128
D_HID = 128
N_CLS = 16

NC = 2
NS = 16
L = 16
NW = NC * NS
EPW = N_EDGES // NW
NPAD = 10240
RPT = NPAD // NS


def _make_edge_kernel(D, CH, UNROLL):
    W = D + L
    NCHD = D // L
    NCHUNK = EPW // CH
    mesh = plsc.VectorSubcoreMesh(
        core_axis_name="c", subcore_axis_name="s",
        num_cores=NC, num_subcores=NS)

    def body(xl_hbm, xr_hbm, src_hbm, dst_hbm, att_hbm, acc_hbm,
             idxs0, idxd0, xls0, xrs0, stage0,
             idxs1, idxd1, xls1, xrs1, stage1,
             attv, accsh, si0, sl0, sr0, si1, sl1, sr1):
        cid = lax.axis_index("c")
        sid = lax.axis_index("s")
        wid = sid * NC + cid

        bufs = [(idxs0, idxd0, xls0, xrs0, stage0, si0, sl0, sr0),
                (idxs1, idxd1, xls1, xrs1, stage1, si1, sl1, sr1)]

        zero = jnp.zeros((L,), jnp.float32)

        def zrow(r, carry):
            for c in range(W // L):
                stage0[r, pl.ds(c * L, L)] = zero
            return carry
        lax.fori_loop(0, CH, zrow, 0)

        for b in range(RPT // CH):
            pltpu.sync_copy(stage0, accsh.at[pl.ds(sid * RPT + b * CH, CH)])
        pltpu.sync_copy(att_hbm, attv)
        plsc.subcore_barrier()

        att_c = [attv[pl.ds(c * L, L)] for c in range(NCHD)]

        def sd_start(i, b):
            idxs, idxd, _, _, _, si, _, _ = bufs[b]
            row = wid * NCHUNK + i
            pltpu.make_async_copy(src_hbm.at[row], idxs, si).start()
            pltpu.make_async_copy(dst_hbm.at[row], idxd, si).start()

        def sd_wait(b):
            idxs, idxd, _, _, _, si, _, _ = bufs[b]
            pltpu.make_async_copy(src_hbm.at[0], idxs, si).wait()
            pltpu.make_async_copy(dst_hbm.at[0], idxd, si).wait()

        def g_start(b):
            idxs, idxd, xls, xrs, _, _, sl, sr = bufs[b]
            pltpu.make_async_copy(xl_hbm.at[idxs], xls, sl).start()
            pltpu.make_async_copy(xr_hbm.at[idxd], xrs, sr).start()

        def g_wait(b):
            idxs, idxd, xls, xrs, _, _, sl, sr = bufs[b]
            pltpu.make_async_copy(xl_hbm.at[idxs], xls, sl).wait()
            pltpu.make_async_copy(xr_hbm.at[idxd], xrs, sr).wait()

        def process(b):
            _, idxd, xls, xrs, stage, _, _, _ = bufs[b]

            @plsc.parallel_loop(0, CH, unroll=UNROLL)
            def edge(e):
                acc = jnp.zeros((L,), jnp.float32)
                a_ch = []
                for c in range(NCHD):
                    a = xls[e, pl.ds(c * L, L)]
                    b_ = xrs[e, pl.ds(c * L, L)]
                    s = a + b_
                    lr = jnp.maximum(s, 0.2 * s)
                    acc = acc + lr * att_c[c]
                    a_ch.append(a)
                tot = plsc.cumsum(acc)[L - 1]
                wv = jnp.exp(jnp.broadcast_to(tot, (L,)))
                for c in range(NCHD):
                    stage[e, pl.ds(c * L, L)] = a_ch[c] * wv
                stage[e, pl.ds(D, L)] = wv

            pltpu.sync_copy(stage, accsh.at[idxd], add=True)

        sd_start(0, 0)
        sd_wait(0)
        g_start(0)
        sd_start(1, 1)

        def chunk2(i2, carry):
            for b in range(2):
                i = i2 * 2 + b
                g_wait(b)
                sd_wait(1 - b)
                g_start(1 - b)
                process(b)
                sd_start(i + 2, b)
            return carry
        if NCHUNK % 2 == 0:
            lax.fori_loop(0, (NCHUNK - 2) // 2, chunk2, 0)
            tail = [(0, None), (1, None)]
        else:
            lax.fori_loop(0, (NCHUNK - 3) // 2, chunk2, 0)
            tail = [(0, NCHUNK - 1), (1, None), (0, None)]
        for k, (b, nxt) in enumerate(tail):
            g_wait(b)
            if k + 1 < len(tail):
                sd_wait(1 - b)
                g_start(1 - b)
            process(b)
            if nxt is not None:
                sd_start(nxt, b)

        plsc.subcore_barrier()
        r0 = sid * RPT
        pltpu.sync_copy(accsh.at[pl.ds(r0, RPT)],
                        acc_hbm.at[cid, pl.ds(r0, RPT)])

    return pl.kernel(
        body,
        out_type=jax.ShapeDtypeStruct((NC, NPAD, W), jnp.float32),
        mesh=mesh,
        scratch_types=[
            pltpu.VMEM((CH,), jnp.int32),
            pltpu.VMEM((CH,), jnp.int32),
            pltpu.VMEM((CH, D), jnp.float32),
            pltpu.VMEM((CH, D), jnp.float32),
            pltpu.VMEM((CH, W), jnp.float32),
            pltpu.VMEM((CH,), jnp.int32),
            pltpu.VMEM((CH,), jnp.int32),
            pltpu.VMEM((CH, D), jnp.float32),
            pltpu.VMEM((CH, D), jnp.float32),
            pltpu.VMEM((CH, W), jnp.float32),
            pltpu.VMEM((D,), jnp.float32),
            pltpu.VMEM_SHARED((NPAD, W), jnp.float32),
            pltpu.SemaphoreType.DMA,
            pltpu.SemaphoreType.DMA,
            pltpu.SemaphoreType.DMA,
            pltpu.SemaphoreType.DMA,
            pltpu.SemaphoreType.DMA,
            pltpu.SemaphoreType.DMA,
        ],
        compiler_params=pltpu.CompilerParams(
            needs_layout_passes=False, use_tc_tiling_on_sc=False),
    )


_edge_conv1 = _make_edge_kernel(D_HID, 40, 4)
_edge_conv2 = _make_edge_kernel(N_CLS, 80, 8)

BR = 1000
_GRID = N_NODES // BR
_HI = lax.Precision.HIGHEST


def _mmT(a, w):
    return lax.dot_general(a, w, (((1,), (1,)), ((), ())), precision=_HI)


def _tc_pre_body(x_ref, wl_ref, bl_ref, wr_ref, br_ref, wlin_ref, blin_ref,
                 xl_ref, xr_ref, lin_ref):
    xb = x_ref[...]
    xl_ref[...] = _mmT(xb, wl_ref[...]) + bl_ref[...]
    xr_ref[...] = _mmT(xb, wr_ref[...]) + br_ref[...]
    lin_ref[...] = _mmT(xb, wlin_ref[...]) + blin_ref[...]


def _tc_mid_body(acc_ref, xl_ref, xr_ref, lin_ref, att_ref, bias_ref,
                 blin_ref, wl2_ref, bl2_ref, wr2_ref, br2_ref,
                 h_ref, xl2_ref, xr2_ref):
    xl = xl_ref[...]
    xr = xr_ref[...]
    s = xl + xr
    lr = jnp.maximum(s, 0.2 * s)
    es = jnp.sum(lr * att_ref[...], axis=1, keepdims=True)
    es = jnp.exp(es)
    acc = acc_ref[...]
    num = acc[0, :, :D_HID] + acc[1, :, :D_HID] + es * xl
    den = acc[0, :, D_HID:D_HID + 1] + acc[1, :, D_HID:D_HID + 1] + es + 1e-16
    conv = num / den + bias_ref[...]
    z = conv + lin_ref[...] + blin_ref[...]
    h = jnp.where(z > 0, z, jnp.exp(z) - 1.0)
    h_ref[...] = h
    xl2_ref[...] = _mmT(h, wl2_ref[...]) + bl2_ref[...]
    xr2_ref[...] = _mmT(h, wr2_ref[...]) + br2_ref[...]


def _tc_post_body(acc_ref, xl2_ref, xr2_ref, h_ref, att_ref, bias_ref,
                  wlin_ref, blin_ref, out_ref):
    xl = xl2_ref[...]
    xr = xr2_ref[...]
    s = xl + xr
    lr = jnp.maximum(s, 0.2 * s)
    es = jnp.exp(jnp.sum(lr * att_ref[...], axis=1, keepdims=True))
    acc = acc_ref[...]
    num = acc[0, :, :N_CLS] + acc[1, :, :N_CLS] + es * xl
    den = acc[0, :, N_CLS:N_CLS + 1] + acc[1, :, N_CLS:N_CLS + 1] + es + 1e-16
    o = num / den + bias_ref[...] + _mmT(h_ref[...], wlin_ref[...]) + blin_ref[...]
    m = jnp.max(o, axis=1, keepdims=True)
    o = o - m
    out_ref[...] = o - jnp.log(jnp.sum(jnp.exp(o), axis=1, keepdims=True))


def _full(shape):
    return pl.BlockSpec(shape, lambda i: tuple(0 for _ in shape))


def _rows(d):
    return pl.BlockSpec((BR, d), lambda i: (i, 0))


def _acc_spec(w):
    return pl.BlockSpec((NC, BR, w), lambda i: (0, i, 0))


def kernel(x, edge_index, Wl1, bl1, Wr1, br1, att1, bias1, W_lin1, b_lin1,
           Wl2, bl2, Wr2, br2, att2, bias2, W_lin2, b_lin2):
    src = edge_index[0].astype(jnp.int32)
    dst = edge_index[1].astype(jnp.int32)
    src40 = src.reshape(-1, 40)
    dst40 = dst.reshape(-1, 40)
    src80 = src.reshape(-1, 80)
    dst80 = dst.reshape(-1, 80)


    xl1, xr1, lin1 = pl.pallas_call(
        _tc_pre_body,
        grid=(_GRID,),
        in_specs=[_rows(D_IN), _full((D_HID, D_IN)), _full((1, D_HID)),
                  _full((D_HID, D_IN)), _full((1, D_HID)),
                  _full((D_HID, D_IN)), _full((1, D_HID))],
        out_specs=[_rows(D_HID), _rows(D_HID), _rows(D_HID)],
        out_shape=[jax.ShapeDtypeStruct((N_NODES, D_HID), jnp.float32)] * 3,
    )(x, Wl1, bl1.reshape(1, -1), Wr1, br1.reshape(1, -1),
      W_lin1, b_lin1.reshape(1, -1))

    acc1 = _edge_conv1(xl1, xr1, src40, dst40, att1)

    h, xl2, xr2 = pl.pallas_call(
        _tc_mid_body,
        grid=(_GRID,),
        in_specs=[_acc_spec(D_HID + L), _rows(D_HID), _rows(D_HID),
                  _rows(D_HID), _full((1, D_HID)), _full((1, D_HID)),
                  _full((1, D_HID)), _full((N_CLS, D_HID)), _full((1, N_CLS)),
                  _full((N_CLS, D_HID)), _full((1, N_CLS))],
        out_specs=[_rows(D_HID), _rows(N_CLS), _rows(N_CLS)],
        out_shape=[jax.ShapeDtypeStruct((N_NODES, D_HID), jnp.float32),
                   jax.ShapeDtypeStruct((N_NODES, N_CLS), jnp.float32),
                   jax.ShapeDtypeStruct((N_NODES, N_CLS), jnp.float32)],
    )(acc1, xl1, xr1, lin1, att1.reshape(1, -1), bias1.reshape(1, -1),
      b_lin1.reshape(1, -1), Wl2, bl2.reshape(1, -1), Wr2, br2.reshape(1, -1))

    acc2 = _edge_conv2(xl2, xr2, src80, dst80, att2)

    out = pl.pallas_call(
        _tc_post_body,
        grid=(_GRID,),
        in_specs=[_acc_spec(N_CLS + L), _rows(N_CLS), _rows(N_CLS),
                  _rows(D_HID), _full((1, N_CLS)), _full((1, N_CLS)),
                  _full((N_CLS, D_HID)), _full((1, N_CLS))],
        out_specs=[_rows(N_CLS)],
        out_shape=[jax.ShapeDtypeStruct((N_NODES, N_CLS), jnp.float32)],
    )(acc2, xl2, xr2, h, att2.reshape(1, -1), bias2.reshape(1, -1),
      W_lin2, b_lin2.reshape(1, -1))[0]

    return (out, edge_index)


# speedup vs baseline: 1.3264x; 1.0053x over previous
import jax
import jax.numpy as jnp
from jax import lax
from jax.experimental import pallas as pl
from jax.experimental.pallas import tpu as pltpu
from jax.experimental.pallas import tpu_sc as plsc

N_NODES = 10000
N_EDGES = 320000
D_IN = 128
D_HID = 128
N_CLS = 16

NC = 2
NS = 16
L = 16
NW = NC * NS
EPW = N_EDGES // NW
NPAD = 10240
RPT = NPAD // NS


def _make_edge_kernel(D, CH, UNROLL):
    W = D + L
    NCHD = D // L
    NCHUNK = EPW // CH
    mesh = plsc.VectorSubcoreMesh(
        core_axis_name="c", subcore_axis_name="s",
        num_cores=NC, num_subcores=NS)

    def body(xl_hbm, xr_hbm, src_hbm, dst_hbm, att_hbm, acc_hbm,
             idxs0, idxd0, xls0, xrs0, stage0,
             idxs1, idxd1, xls1, xrs1, stage1,
             attv, accsh, si0, sl0, sr0, si1, sl1, sr1):
        cid = lax.axis_index("c")
        sid = lax.axis_index("s")
        wid = sid * NC + cid

        bufs = [(idxs0, idxd0, xls0, xrs0, stage0, si0, sl0, sr0),
                (idxs1, idxd1, xls1, xrs1, stage1, si1, sl1, sr1)]

        zero = jnp.zeros((L,), jnp.float32)

        def zrow(r, carry):
            for c in range(W // L):
                stage0[r, pl.ds(c * L, L)] = zero
            return carry
        lax.fori_loop(0, CH, zrow, 0)

        for b in range(RPT // CH):
            pltpu.sync_copy(stage0, accsh.at[pl.ds(sid * RPT + b * CH, CH)])
        pltpu.sync_copy(att_hbm, attv)
        plsc.subcore_barrier()

        att_c = [attv[pl.ds(c * L, L)] for c in range(NCHD)]

        def sd_start(i, b):
            idxs, idxd, _, _, _, si, _, _ = bufs[b]
            row = wid * NCHUNK + i
            pltpu.make_async_copy(src_hbm.at[row], idxs, si).start()
            pltpu.make_async_copy(dst_hbm.at[row], idxd, si).start()

        def sd_wait(b):
            idxs, idxd, _, _, _, si, _, _ = bufs[b]
            pltpu.make_async_copy(src_hbm.at[0], idxs, si).wait()
            pltpu.make_async_copy(dst_hbm.at[0], idxd, si).wait()

        def g_start(b):
            idxs, idxd, xls, xrs, _, _, sl, sr = bufs[b]
            pltpu.make_async_copy(xl_hbm.at[idxs], xls, sl).start()
            pltpu.make_async_copy(xr_hbm.at[idxd], xrs, sr).start()

        def g_wait(b):
            idxs, idxd, xls, xrs, _, _, sl, sr = bufs[b]
            pltpu.make_async_copy(xl_hbm.at[idxs], xls, sl).wait()
            pltpu.make_async_copy(xr_hbm.at[idxd], xrs, sr).wait()

        def process(b):
            _, idxd, xls, xrs, stage, _, _, _ = bufs[b]

            @plsc.parallel_loop(0, CH, unroll=UNROLL)
            def edge(e):
                acc = jnp.zeros((L,), jnp.float32)
                a_ch = []
                for c in range(NCHD):
                    a = xls[e, pl.ds(c * L, L)]
                    b_ = xrs[e, pl.ds(c * L, L)]
                    s = a + b_
                    lr = jnp.maximum(s, 0.2 * s)
                    acc = acc + lr * att_c[c]
                    a_ch.append(a)
                tot = plsc.cumsum(acc)[L - 1]
                wv = jnp.exp(jnp.broadcast_to(tot, (L,)))
                for c in range(NCHD):
                    stage[e, pl.ds(c * L, L)] = a_ch[c] * wv
                stage[e, pl.ds(D, L)] = wv

            pltpu.sync_copy(stage, accsh.at[idxd], add=True)

        sd_start(0, 0)
        sd_wait(0)
        g_start(0)
        sd_start(1, 1)

        def chunk2(i2, carry):
            for b in range(2):
                i = i2 * 2 + b
                g_wait(b)
                sd_wait(1 - b)
                g_start(1 - b)
                process(b)
                sd_start(i + 2, b)
            return carry
        if NCHUNK % 2 == 0:
            lax.fori_loop(0, (NCHUNK - 2) // 2, chunk2, 0)
            tail = [(0, None), (1, None)]
        else:
            lax.fori_loop(0, (NCHUNK - 3) // 2, chunk2, 0)
            tail = [(0, NCHUNK - 1), (1, None), (0, None)]
        for k, (b, nxt) in enumerate(tail):
            g_wait(b)
            if k + 1 < len(tail):
                sd_wait(1 - b)
                g_start(1 - b)
            process(b)
            if nxt is not None:
                sd_start(nxt, b)

        plsc.subcore_barrier()
        r0 = sid * RPT
        pltpu.sync_copy(accsh.at[pl.ds(r0, RPT)],
                        acc_hbm.at[cid, pl.ds(r0, RPT)])

    return pl.kernel(
        body,
        out_type=jax.ShapeDtypeStruct((NC, NPAD, W), jnp.float32),
        mesh=mesh,
        scratch_types=[
            pltpu.VMEM((CH,), jnp.int32),
            pltpu.VMEM((CH,), jnp.int32),
            pltpu.VMEM((CH, D), jnp.float32),
            pltpu.VMEM((CH, D), jnp.float32),
            pltpu.VMEM((CH, W), jnp.float32),
            pltpu.VMEM((CH,), jnp.int32),
            pltpu.VMEM((CH,), jnp.int32),
            pltpu.VMEM((CH, D), jnp.float32),
            pltpu.VMEM((CH, D), jnp.float32),
            pltpu.VMEM((CH, W), jnp.float32),
            pltpu.VMEM((D,), jnp.float32),
            pltpu.VMEM_SHARED((NPAD, W), jnp.float32),
            pltpu.SemaphoreType.DMA,
            pltpu.SemaphoreType.DMA,
            pltpu.SemaphoreType.DMA,
            pltpu.SemaphoreType.DMA,
            pltpu.SemaphoreType.DMA,
            pltpu.SemaphoreType.DMA,
        ],
        compiler_params=pltpu.CompilerParams(
            needs_layout_passes=False, use_tc_tiling_on_sc=False),
    )


_edge_conv1 = _make_edge_kernel(D_HID, 40, 2)
_edge_conv2 = _make_edge_kernel(N_CLS, 80, 8)

BR = 1000
_GRID = N_NODES // BR
_HI = lax.Precision.HIGHEST


def _mmT(a, w):
    return lax.dot_general(a, w, (((1,), (1,)), ((), ())), precision=_HI)


def _tc_pre_body(x_ref, wl_ref, bl_ref, wr_ref, br_ref, wlin_ref, blin_ref,
                 xl_ref, xr_ref, lin_ref):
    xb = x_ref[...]
    xl_ref[...] = _mmT(xb, wl_ref[...]) + bl_ref[...]
    xr_ref[...] = _mmT(xb, wr_ref[...]) + br_ref[...]
    lin_ref[...] = _mmT(xb, wlin_ref[...]) + blin_ref[...]


def _tc_mid_body(acc_ref, xl_ref, xr_ref, lin_ref, att_ref, bias_ref,
                 blin_ref, wl2_ref, bl2_ref, wr2_ref, br2_ref,
                 h_ref, xl2_ref, xr2_ref):
    xl = xl_ref[...]
    xr = xr_ref[...]
    s = xl + xr
    lr = jnp.maximum(s, 0.2 * s)
    es = jnp.sum(lr * att_ref[...], axis=1, keepdims=True)
    es = jnp.exp(es)
    acc = acc_ref[...]
    num = acc[0, :, :D_HID] + acc[1, :, :D_HID] + es * xl
    den = acc[0, :, D_HID:D_HID + 1] + acc[1, :, D_HID:D_HID + 1] + es + 1e-16
    conv = num / den + bias_ref[...]
    z = conv + lin_ref[...] + blin_ref[...]
    h = jnp.where(z > 0, z, jnp.exp(z) - 1.0)
    h_ref[...] = h
    xl2_ref[...] = _mmT(h, wl2_ref[...]) + bl2_ref[...]
    xr2_ref[...] = _mmT(h, wr2_ref[...]) + br2_ref[...]


def _tc_post_body(acc_ref, xl2_ref, xr2_ref, h_ref, att_ref, bias_ref,
                  wlin_ref, blin_ref, out_ref):
    xl = xl2_ref[...]
    xr = xr2_ref[...]
    s = xl + xr
    lr = jnp.maximum(s, 0.2 * s)
    es = jnp.exp(jnp.sum(lr * att_ref[...], axis=1, keepdims=True))
    acc = acc_ref[...]
    num = acc[0, :, :N_CLS] + acc[1, :, :N_CLS] + es * xl
    den = acc[0, :, N_CLS:N_CLS + 1] + acc[1, :, N_CLS:N_CLS + 1] + es + 1e-16
    o = num / den + bias_ref[...] + _mmT(h_ref[...], wlin_ref[...]) + blin_ref[...]
    m = jnp.max(o, axis=1, keepdims=True)
    o = o - m
    out_ref[...] = o - jnp.log(jnp.sum(jnp.exp(o), axis=1, keepdims=True))


def _full(shape):
    return pl.BlockSpec(shape, lambda i: tuple(0 for _ in shape))


def _rows(d):
    return pl.BlockSpec((BR, d), lambda i: (i, 0))


def _acc_spec(w):
    return pl.BlockSpec((NC, BR, w), lambda i: (0, i, 0))


def kernel(x, edge_index, Wl1, bl1, Wr1, br1, att1, bias1, W_lin1, b_lin1,
           Wl2, bl2, Wr2, br2, att2, bias2, W_lin2, b_lin2):
    src = edge_index[0].astype(jnp.int32)
    dst = edge_index[1].astype(jnp.int32)
    src40 = src.reshape(-1, 40)
    dst40 = dst.reshape(-1, 40)
    src80 = src.reshape(-1, 80)
    dst80 = dst.reshape(-1, 80)


    xl1, xr1, lin1 = pl.pallas_call(
        _tc_pre_body,
        grid=(_GRID,),
        in_specs=[_rows(D_IN), _full((D_HID, D_IN)), _full((1, D_HID)),
                  _full((D_HID, D_IN)), _full((1, D_HID)),
                  _full((D_HID, D_IN)), _full((1, D_HID))],
        out_specs=[_rows(D_HID), _rows(D_HID), _rows(D_HID)],
        out_shape=[jax.ShapeDtypeStruct((N_NODES, D_HID), jnp.float32)] * 3,
    )(x, Wl1, bl1.reshape(1, -1), Wr1, br1.reshape(1, -1),
      W_lin1, b_lin1.reshape(1, -1))

    acc1 = _edge_conv1(xl1, xr1, src40, dst40, att1)

    h, xl2, xr2 = pl.pallas_call(
        _tc_mid_body,
        grid=(_GRID,),
        in_specs=[_acc_spec(D_HID + L), _rows(D_HID), _rows(D_HID),
                  _rows(D_HID), _full((1, D_HID)), _full((1, D_HID)),
                  _full((1, D_HID)), _full((N_CLS, D_HID)), _full((1, N_CLS)),
                  _full((N_CLS, D_HID)), _full((1, N_CLS))],
        out_specs=[_rows(D_HID), _rows(N_CLS), _rows(N_CLS)],
        out_shape=[jax.ShapeDtypeStruct((N_NODES, D_HID), jnp.float32),
                   jax.ShapeDtypeStruct((N_NODES, N_CLS), jnp.float32),
                   jax.ShapeDtypeStruct((N_NODES, N_CLS), jnp.float32)],
    )(acc1, xl1, xr1, lin1, att1.reshape(1, -1), bias1.reshape(1, -1),
      b_lin1.reshape(1, -1), Wl2, bl2.reshape(1, -1), Wr2, br2.reshape(1, -1))

    acc2 = _edge_conv2(xl2, xr2, src80, dst80, att2)

    out = pl.pallas_call(
        _tc_post_body,
        grid=(_GRID,),
        in_specs=[_acc_spec(N_CLS + L), _rows(N_CLS), _rows(N_CLS),
                  _rows(D_HID), _full((1, N_CLS)), _full((1, N_CLS)),
                  _full((N_CLS, D_HID)), _full((1, N_CLS))],
        out_specs=[_rows(N_CLS)],
        out_shape=[jax.ShapeDtypeStruct((N_NODES, N_CLS), jnp.float32)],
    )(acc2, xl2, xr2, h, att2.reshape(1, -1), bias2.reshape(1, -1),
      W_lin2, b_lin2.reshape(1, -1))[0]

    return (out, edge_index)


# speedup vs baseline: 1.3584x; 1.0241x over previous
import jax
import jax.numpy as jnp
from jax import lax
from jax.experimental import pallas as pl
from jax.experimental.pallas import tpu as pltpu
from jax.experimental.pallas import tpu_sc as plsc

N_NODES = 10000
N_EDGES = 320000
D_IN = 128
D_HID = 128
N_CLS = 16

NC = 2
NS = 16
L = 16
NW = NC * NS
EPW = N_EDGES // NW
NPAD = 10240
RPT = NPAD // NS


def _make_edge_kernel(D, CH, UNROLL):
    W = D + L
    NCHD = D // L
    NCHUNK = EPW // CH
    mesh = plsc.VectorSubcoreMesh(
        core_axis_name="c", subcore_axis_name="s",
        num_cores=NC, num_subcores=NS)

    def body(xl_hbm, xr_hbm, src_hbm, dst_hbm, att_hbm, acc_hbm,
             idxs0, idxd0, xls0, xrs0, stage0,
             idxs1, idxd1, xls1, xrs1, stage1,
             attv, accsh, si0, sl0, sr0, si1, sl1, sr1):
        cid = lax.axis_index("c")
        sid = lax.axis_index("s")
        wid = sid * NC + cid

        bufs = [(idxs0, idxd0, xls0, xrs0, stage0, si0, sl0, sr0),
                (idxs1, idxd1, xls1, xrs1, stage1, si1, sl1, sr1)]

        zero = jnp.zeros((L,), jnp.float32)

        def zrow(r, carry):
            for c in range(W // L):
                stage0[r, pl.ds(c * L, L)] = zero
            return carry
        lax.fori_loop(0, CH, zrow, 0)

        for b in range(RPT // CH):
            pltpu.sync_copy(stage0, accsh.at[pl.ds(sid * RPT + b * CH, CH)])
        pltpu.sync_copy(att_hbm, attv)
        plsc.subcore_barrier()

        att_c = [attv[pl.ds(c * L, L)] for c in range(NCHD)]

        def sd_start(i, b):
            idxs, idxd, _, _, _, si, _, _ = bufs[b]
            row = wid * NCHUNK + i
            pltpu.make_async_copy(src_hbm.at[row], idxs, si).start()
            pltpu.make_async_copy(dst_hbm.at[row], idxd, si).start()

        def sd_wait(b):
            idxs, idxd, _, _, _, si, _, _ = bufs[b]
            pltpu.make_async_copy(src_hbm.at[0], idxs, si).wait()
            pltpu.make_async_copy(dst_hbm.at[0], idxd, si).wait()

        def g_start(b):
            idxs, idxd, xls, xrs, _, _, sl, sr = bufs[b]
            pltpu.make_async_copy(xl_hbm.at[idxs], xls, sl).start()
            pltpu.make_async_copy(xr_hbm.at[idxd], xrs, sr).start()

        def g_wait(b):
            idxs, idxd, xls, xrs, _, _, sl, sr = bufs[b]
            pltpu.make_async_copy(xl_hbm.at[idxs], xls, sl).wait()
            pltpu.make_async_copy(xr_hbm.at[idxd], xrs, sr).wait()

        def process(b):
            _, idxd, xls, xrs, stage, _, _, _ = bufs[b]

            @plsc.parallel_loop(0, CH, unroll=UNROLL)
            def edge(e):
                acc = jnp.zeros((L,), jnp.float32)
                a_ch = []
                for c in range(NCHD):
                    a = xls[e, pl.ds(c * L, L)]
                    b_ = xrs[e, pl.ds(c * L, L)]
                    s = a + b_
                    lr = jnp.maximum(s, 0.2 * s)
                    acc = acc + lr * att_c[c]
                    a_ch.append(a)
                tot = plsc.cumsum(acc)[L - 1]
                wv = jnp.exp(jnp.broadcast_to(tot, (L,)))
                for c in range(NCHD):
                    stage[e, pl.ds(c * L, L)] = a_ch[c] * wv
                stage[e, pl.ds(D, L)] = wv

            pltpu.sync_copy(stage, accsh.at[idxd], add=True)

        sd_start(0, 0)
        sd_wait(0)
        g_start(0)
        sd_start(1, 1)

        def chunk2(i2, carry):
            for b in range(2):
                i = i2 * 2 + b
                g_wait(b)
                sd_wait(1 - b)
                g_start(1 - b)
                process(b)
                sd_start(i + 2, b)
            return carry
        if NCHUNK % 2 == 0:
            lax.fori_loop(0, (NCHUNK - 2) // 2, chunk2, 0)
            tail = [(0, None), (1, None)]
        else:
            lax.fori_loop(0, (NCHUNK - 3) // 2, chunk2, 0)
            tail = [(0, NCHUNK - 1), (1, None), (0, None)]
        for k, (b, nxt) in enumerate(tail):
            g_wait(b)
            if k + 1 < len(tail):
                sd_wait(1 - b)
                g_start(1 - b)
            process(b)
            if nxt is not None:
                sd_start(nxt, b)

        plsc.subcore_barrier()
        r0 = sid * RPT
        pltpu.sync_copy(accsh.at[pl.ds(r0, RPT)],
                        acc_hbm.at[cid, pl.ds(r0, RPT)])

    return pl.kernel(
        body,
        out_type=jax.ShapeDtypeStruct((NC, NPAD, W), jnp.float32),
        mesh=mesh,
        scratch_types=[
            pltpu.VMEM((CH,), jnp.int32),
            pltpu.VMEM((CH,), jnp.int32),
            pltpu.VMEM((CH, D), jnp.float32),
            pltpu.VMEM((CH, D), jnp.float32),
            pltpu.VMEM((CH, W), jnp.float32),
            pltpu.VMEM((CH,), jnp.int32),
            pltpu.VMEM((CH,), jnp.int32),
            pltpu.VMEM((CH, D), jnp.float32),
            pltpu.VMEM((CH, D), jnp.float32),
            pltpu.VMEM((CH, W), jnp.float32),
            pltpu.VMEM((D,), jnp.float32),
            pltpu.VMEM_SHARED((NPAD, W), jnp.float32),
            pltpu.SemaphoreType.DMA,
            pltpu.SemaphoreType.DMA,
            pltpu.SemaphoreType.DMA,
            pltpu.SemaphoreType.DMA,
            pltpu.SemaphoreType.DMA,
            pltpu.SemaphoreType.DMA,
        ],
        compiler_params=pltpu.CompilerParams(
            needs_layout_passes=False, use_tc_tiling_on_sc=False),
    )


_edge_conv1 = _make_edge_kernel(D_HID, 40, 3)
_edge_conv2 = _make_edge_kernel(N_CLS, 80, 8)

BR = 1000
_GRID = N_NODES // BR
_HI = lax.Precision.HIGHEST


def _mmT(a, w):
    return lax.dot_general(a, w, (((1,), (1,)), ((), ())), precision=_HI)


def _tc_pre_body(x_ref, wl_ref, bl_ref, wr_ref, br_ref, wlin_ref, blin_ref,
                 xl_ref, xr_ref, lin_ref):
    xb = x_ref[...]
    xl_ref[...] = _mmT(xb, wl_ref[...]) + bl_ref[...]
    xr_ref[...] = _mmT(xb, wr_ref[...]) + br_ref[...]
    lin_ref[...] = _mmT(xb, wlin_ref[...]) + blin_ref[...]


def _tc_mid_body(acc_ref, xl_ref, xr_ref, lin_ref, att_ref, bias_ref,
                 blin_ref, wl2_ref, bl2_ref, wr2_ref, br2_ref,
                 h_ref, xl2_ref, xr2_ref):
    xl = xl_ref[...]
    xr = xr_ref[...]
    s = xl + xr
    lr = jnp.maximum(s, 0.2 * s)
    es = jnp.sum(lr * att_ref[...], axis=1, keepdims=True)
    es = jnp.exp(es)
    acc = acc_ref[...]
    num = acc[0, :, :D_HID] + acc[1, :, :D_HID] + es * xl
    den = acc[0, :, D_HID:D_HID + 1] + acc[1, :, D_HID:D_HID + 1] + es + 1e-16
    conv = num / den + bias_ref[...]
    z = conv + lin_ref[...] + blin_ref[...]
    h = jnp.where(z > 0, z, jnp.exp(z) - 1.0)
    h_ref[...] = h
    xl2_ref[...] = _mmT(h, wl2_ref[...]) + bl2_ref[...]
    xr2_ref[...] = _mmT(h, wr2_ref[...]) + br2_ref[...]


def _tc_post_body(acc_ref, xl2_ref, xr2_ref, h_ref, att_ref, bias_ref,
                  wlin_ref, blin_ref, out_ref):
    xl = xl2_ref[...]
    xr = xr2_ref[...]
    s = xl + xr
    lr = jnp.maximum(s, 0.2 * s)
    es = jnp.exp(jnp.sum(lr * att_ref[...], axis=1, keepdims=True))
    acc = acc_ref[...]
    num = acc[0, :, :N_CLS] + acc[1, :, :N_CLS] + es * xl
    den = acc[0, :, N_CLS:N_CLS + 1] + acc[1, :, N_CLS:N_CLS + 1] + es + 1e-16
    o = num / den + bias_ref[...] + _mmT(h_ref[...], wlin_ref[...]) + blin_ref[...]
    m = jnp.max(o, axis=1, keepdims=True)
    o = o - m
    out_ref[...] = o - jnp.log(jnp.sum(jnp.exp(o), axis=1, keepdims=True))


def _full(shape):
    return pl.BlockSpec(shape, lambda i: tuple(0 for _ in shape))


def _rows(d):
    return pl.BlockSpec((BR, d), lambda i: (i, 0))


def _acc_spec(w):
    return pl.BlockSpec((NC, BR, w), lambda i: (0, i, 0))


def kernel(x, edge_index, Wl1, bl1, Wr1, br1, att1, bias1, W_lin1, b_lin1,
           Wl2, bl2, Wr2, br2, att2, bias2, W_lin2, b_lin2):
    src = edge_index[0].astype(jnp.int32)
    dst = edge_index[1].astype(jnp.int32)
    src40 = src.reshape(-1, 40)
    dst40 = dst.reshape(-1, 40)
    src80 = src.reshape(-1, 80)
    dst80 = dst.reshape(-1, 80)


    xl1, xr1, lin1 = pl.pallas_call(
        _tc_pre_body,
        grid=(_GRID,),
        in_specs=[_rows(D_IN), _full((D_HID, D_IN)), _full((1, D_HID)),
                  _full((D_HID, D_IN)), _full((1, D_HID)),
                  _full((D_HID, D_IN)), _full((1, D_HID))],
        out_specs=[_rows(D_HID), _rows(D_HID), _rows(D_HID)],
        out_shape=[jax.ShapeDtypeStruct((N_NODES, D_HID), jnp.float32)] * 3,
    )(x, Wl1, bl1.reshape(1, -1), Wr1, br1.reshape(1, -1),
      W_lin1, b_lin1.reshape(1, -1))

    acc1 = _edge_conv1(xl1, xr1, src40, dst40, att1)

    h, xl2, xr2 = pl.pallas_call(
        _tc_mid_body,
        grid=(_GRID,),
        in_specs=[_acc_spec(D_HID + L), _rows(D_HID), _rows(D_HID),
                  _rows(D_HID), _full((1, D_HID)), _full((1, D_HID)),
                  _full((1, D_HID)), _full((N_CLS, D_HID)), _full((1, N_CLS)),
                  _full((N_CLS, D_HID)), _full((1, N_CLS))],
        out_specs=[_rows(D_HID), _rows(N_CLS), _rows(N_CLS)],
        out_shape=[jax.ShapeDtypeStruct((N_NODES, D_HID), jnp.float32),
                   jax.ShapeDtypeStruct((N_NODES, N_CLS), jnp.float32),
                   jax.ShapeDtypeStruct((N_NODES, N_CLS), jnp.float32)],
    )(acc1, xl1, xr1, lin1, att1.reshape(1, -1), bias1.reshape(1, -1),
      b_lin1.reshape(1, -1), Wl2, bl2.reshape(1, -1), Wr2, br2.reshape(1, -1))

    acc2 = _edge_conv2(xl2, xr2, src80, dst80, att2)

    out = pl.pallas_call(
        _tc_post_body,
        grid=(_GRID,),
        in_specs=[_acc_spec(N_CLS + L), _rows(N_CLS), _rows(N_CLS),
                  _rows(D_HID), _full((1, N_CLS)), _full((1, N_CLS)),
                  _full((N_CLS, D_HID)), _full((1, N_CLS))],
        out_specs=[_rows(N_CLS)],
        out_shape=[jax.ShapeDtypeStruct((N_NODES, N_CLS), jnp.float32)],
    )(acc2, xl2, xr2, h, att2.reshape(1, -1), bias2.reshape(1, -1),
      W_lin2, b_lin2.reshape(1, -1))[0]

    return (out, edge_index)
